# baseline (speedup 1.0000x reference)
import jax
import jax.numpy as jnp
from jax.experimental import pallas as pl
from jax.experimental.pallas import tpu as pltpu

_BLOCK = 8000


def _mpn_block_kernel(fs_ref, f_ref, sm_ref, w1_ref, w2_ref, w3_ref, b_ref,
                      out_ref):
    acc = jnp.dot(fs_ref[...], w1_ref[...], preferred_element_type=jnp.float32)
    acc = acc + jnp.dot(f_ref[...], w2_ref[...],
                        preferred_element_type=jnp.float32)
    acc = acc + jnp.dot(sm_ref[...], w3_ref[...],
                        preferred_element_type=jnp.float32)
    acc = acc + b_ref[...]
    out_ref[...] = jnp.maximum(acc, 0.0)


def kernel(f_src, f, sum_msg, w1, w2, w3, b):
    e, d_ndata = f_src.shape
    d_edata = f.shape[1]
    d_msg = sum_msg.shape[1]
    block = _BLOCK if e % _BLOCK == 0 else e
    grid = e // block

    return pl.pallas_call(
        _mpn_block_kernel,
        grid=(grid,),
        in_specs=[
            pl.BlockSpec((block, d_ndata), lambda i: (i, 0)),
            pl.BlockSpec((block, d_edata), lambda i: (i, 0)),
            pl.BlockSpec((block, d_msg), lambda i: (i, 0)),
            pl.BlockSpec((d_ndata, d_msg), lambda i: (0, 0)),
            pl.BlockSpec((d_edata, d_msg), lambda i: (0, 0)),
            pl.BlockSpec((d_msg, d_msg), lambda i: (0, 0)),
            pl.BlockSpec((1, d_msg), lambda i: (0, 0)),
        ],
        out_specs=pl.BlockSpec((block, d_msg), lambda i: (i, 0)),
        out_shape=jax.ShapeDtypeStruct((e, d_msg), jnp.float32),
        compiler_params=pltpu.CompilerParams(
            dimension_semantics=("parallel",)),
    )(f_src, f, sum_msg, w1, w2, w3, b)
